# baseline (speedup 1.0000x reference)
import functools

import jax
import jax.numpy as jnp
from jax import lax
from jax.experimental import pallas as pl
from jax.experimental.pallas import tpu as pltpu
from jax.experimental.pallas import tpu_sc as plsc

_VOCAB = 1000000
_N_EMBD = 64
_CHUNK = 200
_BATCH = 1024

_NC = 2
_NS = 16
_NW = _NC * _NS
_ROWS = _BATCH * _CHUNK
_ROWS_PER_W = _ROWS // _NW
_K = 40
_NCHUNK = _ROWS_PER_W // _K
_LANES = 16
_NSL = _N_EMBD // _LANES


@functools.partial(
    pl.kernel,
    mesh=plsc.VectorSubcoreMesh(core_axis_name="c", subcore_axis_name="s"),
    out_type=jax.ShapeDtypeStruct((_ROWS, _N_EMBD), jnp.float32),
    scratch_types=[
        pltpu.VMEM((_K,), jnp.int32),
        pltpu.VMEM((_K, _N_EMBD), jnp.float32),
        pltpu.VMEM((_CHUNK, _N_EMBD), jnp.float32),
        pltpu.SemaphoreType.DMA,
    ],
    compiler_params=pltpu.CompilerParams(use_tc_tiling_on_sc=False),
)
def _emb_lookup(ids_hbm, wte_hbm, wpe_hbm, out_hbm, idx_v, rows_v, wpe_v, sem):
    wid = lax.axis_index("s") * _NC + lax.axis_index("c")
    base = wid * _ROWS_PER_W
    pltpu.sync_copy(wpe_hbm, wpe_v)

    def chunk_body(g, carry):
        rbase = base + g * _K
        pltpu.sync_copy(ids_hbm.at[pl.ds(rbase, _K)], idx_v)
        pltpu.async_copy(wte_hbm.at[idx_v], rows_v, sem).wait()
        off = lax.rem(g * _K, _CHUNK)

        def add_row(r, c2):
            for c in range(_NSL):
                sl = pl.ds(c * _LANES, _LANES)
                rows_v[r, sl] = rows_v[r, sl] + wpe_v[off + r, sl]
            return c2

        lax.fori_loop(0, _K, add_row, 0)
        pltpu.sync_copy(rows_v, out_hbm.at[pl.ds(rbase, _K)])
        return carry

    lax.fori_loop(0, _NCHUNK, chunk_body, 0)


def kernel(input_ids, wte, wpe):
    ids_flat = input_ids.reshape(-1).astype(jnp.int32)
    out = _emb_lookup(ids_flat, wte, wpe)
    return out.reshape(_BATCH, _CHUNK, _N_EMBD)


# speedup vs baseline: 1.2155x; 1.2155x over previous
import functools

import jax
import jax.numpy as jnp
from jax import lax
from jax.experimental import pallas as pl
from jax.experimental.pallas import tpu as pltpu
from jax.experimental.pallas import tpu_sc as plsc

_N_EMBD = 64
_CHUNK = 200
_BATCH = 1024

_NC = 2
_NS = 16
_NW = _NC * _NS
_ROWS = _BATCH * _CHUNK
_ROWS_PER_W = _ROWS // _NW
_K = 40
_NCHUNK = _ROWS_PER_W // _K
_NPAIR = (_NCHUNK - 2) // 2
_LANES = 16
_NSL = _N_EMBD // _LANES


@functools.partial(
    pl.kernel,
    mesh=plsc.VectorSubcoreMesh(core_axis_name="c", subcore_axis_name="s"),
    out_type=jax.ShapeDtypeStruct((_ROWS, _N_EMBD), jnp.float32),
    scratch_types=[
        pltpu.VMEM((_ROWS_PER_W,), jnp.int32),
        pltpu.VMEM((_K, _N_EMBD), jnp.float32),
        pltpu.VMEM((_K, _N_EMBD), jnp.float32),
        pltpu.VMEM((_CHUNK, _N_EMBD), jnp.float32),
        pltpu.SemaphoreType.DMA,
        pltpu.SemaphoreType.DMA,
        pltpu.SemaphoreType.DMA,
        pltpu.SemaphoreType.DMA,
    ],
    compiler_params=pltpu.CompilerParams(use_tc_tiling_on_sc=False),
)
def _emb_lookup(ids_hbm, wte_hbm, wpe_hbm, out_hbm, idx_v, rows_v0, rows_v1,
                wpe_v, sem_g0, sem_g1, sem_o0, sem_o1):
    wid = lax.axis_index("s") * _NC + lax.axis_index("c")
    base = wid * _ROWS_PER_W
    pltpu.sync_copy(ids_hbm.at[pl.ds(base, _ROWS_PER_W)], idx_v)
    pltpu.sync_copy(wpe_hbm, wpe_v)

    def start_gather(g, rows_b, sem_gb):
        pltpu.async_copy(wte_hbm.at[idx_v.at[pl.ds(g * _K, _K)]], rows_b, sem_gb)

    def wait_gather(rows_b, sem_gb):
        pltpu.make_async_copy(wte_hbm.at[pl.ds(0, _K)], rows_b, sem_gb).wait()

    def wait_out(rows_b, sem_ob):
        pltpu.make_async_copy(rows_b, out_hbm.at[pl.ds(0, _K)], sem_ob).wait()

    def add_and_store(g, rows_b, sem_ob):
        off = lax.rem(g * _K, _CHUNK)

        @pl.loop(0, _K, unroll=8)
        def _(r):
            for c in range(_NSL):
                sl = pl.ds(c * _LANES, _LANES)
                rows_b[r, sl] = rows_b[r, sl] + wpe_v[off + r, sl]

        pltpu.async_copy(rows_b, out_hbm.at[pl.ds(base + g * _K, _K)], sem_ob)

    start_gather(0, rows_v0, sem_g0)
    start_gather(1, rows_v1, sem_g1)
    wait_gather(rows_v0, sem_g0)
    add_and_store(0, rows_v0, sem_o0)

    def steady(g, rows_b, sem_gb, sem_ob, rows_o, sem_go, sem_oo):
        wait_out(rows_o, sem_oo)
        start_gather(g + 1, rows_o, sem_go)
        wait_gather(rows_b, sem_gb)
        add_and_store(g, rows_b, sem_ob)

    def pair_body(i, carry):
        g = 2 * i + 1
        steady(g, rows_v1, sem_g1, sem_o1, rows_v0, sem_g0, sem_o0)
        steady(g + 1, rows_v0, sem_g0, sem_o0, rows_v1, sem_g1, sem_o1)
        return carry

    lax.fori_loop(0, _NPAIR, pair_body, 0)

    wait_out(rows_v0, sem_o0)
    wait_gather(rows_v1, sem_g1)
    add_and_store(_NCHUNK - 1, rows_v1, sem_o1)
    wait_out(rows_v1, sem_o1)


def kernel(input_ids, wte, wpe):
    ids_flat = input_ids.reshape(-1).astype(jnp.int32)
    out = _emb_lookup(ids_flat, wte, wpe)
    return out.reshape(_BATCH, _CHUNK, _N_EMBD)


# speedup vs baseline: 1.2853x; 1.0575x over previous
import functools

import jax
import jax.numpy as jnp
from jax import lax
from jax.experimental import pallas as pl
from jax.experimental.pallas import tpu as pltpu
from jax.experimental.pallas import tpu_sc as plsc

_N_EMBD = 64
_CHUNK = 200
_BATCH = 1024

_NC = 2
_NS = 16
_NW = _NC * _NS
_ROWS = _BATCH * _CHUNK
_ROWS_PER_W = _ROWS // _NW
_K = 40
_NCHUNK = _ROWS_PER_W // _K
_NPAIR = (_NCHUNK - 2) // 2
_LANES = 16
_NSL = _N_EMBD // _LANES


@functools.partial(
    pl.kernel,
    mesh=plsc.VectorSubcoreMesh(core_axis_name="c", subcore_axis_name="s"),
    out_type=jax.ShapeDtypeStruct((_ROWS, _N_EMBD), jnp.float32),
    scratch_types=[
        pltpu.VMEM((_ROWS_PER_W,), jnp.int32),
        pltpu.VMEM((_K, _N_EMBD), jnp.float32),
        pltpu.VMEM((_K, _N_EMBD), jnp.float32),
        pltpu.VMEM((_CHUNK, _N_EMBD), jnp.float32),
        pltpu.SemaphoreType.DMA,
        pltpu.SemaphoreType.DMA,
        pltpu.SemaphoreType.DMA,
        pltpu.SemaphoreType.DMA,
    ],
    compiler_params=pltpu.CompilerParams(use_tc_tiling_on_sc=False),
)
def _emb_lookup(ids_hbm, wte_hbm, wpe_hbm, out_hbm, idx_v, rows_v0, rows_v1,
                wpe_v, sem_g0, sem_g1, sem_o0, sem_o1):
    wid = lax.axis_index("s") * _NC + lax.axis_index("c")
    base = wid * _ROWS_PER_W
    pltpu.sync_copy(ids_hbm.at[pl.ds(base, _ROWS_PER_W)], idx_v)
    pltpu.sync_copy(wpe_hbm, wpe_v)

    def start_gather(g, rows_b, sem_gb):
        pltpu.async_copy(wte_hbm.at[idx_v.at[pl.ds(g * _K, _K)]], rows_b, sem_gb)

    def wait_gather(rows_b, sem_gb):
        pltpu.make_async_copy(wte_hbm.at[pl.ds(0, _K)], rows_b, sem_gb).wait()

    def wait_out(rows_b, sem_ob):
        pltpu.make_async_copy(rows_b, out_hbm.at[pl.ds(0, _K)], sem_ob).wait()

    def add_and_store(g, rows_b, sem_ob):
        off = lax.rem(g * _K, _CHUNK)

        @plsc.parallel_loop(0, _K, unroll=8)
        def _(r):
            for c in range(_NSL):
                sl = pl.ds(c * _LANES, _LANES)
                rows_b[r, sl] = rows_b[r, sl] + wpe_v[off + r, sl]

        pltpu.async_copy(rows_b, out_hbm.at[pl.ds(base + g * _K, _K)], sem_ob)

    start_gather(0, rows_v0, sem_g0)
    start_gather(1, rows_v1, sem_g1)
    wait_gather(rows_v0, sem_g0)
    add_and_store(0, rows_v0, sem_o0)

    def steady(g, rows_b, sem_gb, sem_ob, rows_o, sem_go, sem_oo):
        wait_out(rows_o, sem_oo)
        start_gather(g + 1, rows_o, sem_go)
        wait_gather(rows_b, sem_gb)
        add_and_store(g, rows_b, sem_ob)

    def pair_body(i, carry):
        g = 2 * i + 1
        steady(g, rows_v1, sem_g1, sem_o1, rows_v0, sem_g0, sem_o0)
        steady(g + 1, rows_v0, sem_g0, sem_o0, rows_v1, sem_g1, sem_o1)
        return carry

    lax.fori_loop(0, _NPAIR, pair_body, 0)

    wait_out(rows_v0, sem_o0)
    wait_gather(rows_v1, sem_g1)
    add_and_store(_NCHUNK - 1, rows_v1, sem_o1)
    wait_out(rows_v1, sem_o1)


def kernel(input_ids, wte, wpe):
    ids_flat = input_ids.reshape(-1).astype(jnp.int32)
    out = _emb_lookup(ids_flat, wte, wpe)
    return out.reshape(_BATCH, _CHUNK, _N_EMBD)


# speedup vs baseline: 1.3781x; 1.0722x over previous
import functools

import jax
import jax.numpy as jnp
from jax import lax
from jax.experimental import pallas as pl
from jax.experimental.pallas import tpu as pltpu
from jax.experimental.pallas import tpu_sc as plsc

_N_EMBD = 64
_CHUNK = 200
_BATCH = 1024

_NC = 2
_NS = 16
_NW = _NC * _NS
_ROWS = _BATCH * _CHUNK
_ROWS_PER_W = _ROWS // _NW
_K = _CHUNK
_NCHUNK = _ROWS_PER_W // _K
_NBUF = 4
_LANES = 16
_NSL = _N_EMBD // _LANES


@functools.partial(
    pl.kernel,
    mesh=plsc.VectorSubcoreMesh(core_axis_name="c", subcore_axis_name="s"),
    out_type=jax.ShapeDtypeStruct((_ROWS, _N_EMBD), jnp.float32),
    scratch_types=[
        pltpu.VMEM((_ROWS_PER_W,), jnp.int32),
        [pltpu.VMEM((_K, _N_EMBD), jnp.float32) for _ in range(_NBUF)],
        pltpu.VMEM((_CHUNK, _N_EMBD), jnp.float32),
        [pltpu.SemaphoreType.DMA for _ in range(_NBUF)],
        [pltpu.SemaphoreType.DMA for _ in range(_NBUF)],
    ],
    compiler_params=pltpu.CompilerParams(use_tc_tiling_on_sc=False),
)
def _emb_lookup(ids_hbm, wte_hbm, wpe_hbm, out_hbm, idx_v, rows, wpe_v,
                sem_g, sem_o):
    wid = lax.axis_index("s") * _NC + lax.axis_index("c")
    base = wid * _ROWS_PER_W
    pltpu.sync_copy(ids_hbm.at[pl.ds(base, _ROWS_PER_W)], idx_v)
    pltpu.sync_copy(wpe_hbm, wpe_v)

    def start_gather(g, b):
        pltpu.async_copy(wte_hbm.at[idx_v.at[pl.ds(g * _K, _K)]], rows[b],
                         sem_g[b])

    def wait_gather(b):
        pltpu.make_async_copy(wte_hbm.at[pl.ds(0, _K)], rows[b],
                              sem_g[b]).wait()

    def wait_out(b):
        pltpu.make_async_copy(rows[b], out_hbm.at[pl.ds(0, _K)],
                              sem_o[b]).wait()

    def add_and_store(g, b):
        @plsc.parallel_loop(0, _K, unroll=8)
        def _(r):
            for c in range(_NSL):
                sl = pl.ds(c * _LANES, _LANES)
                rows[b][r, sl] = rows[b][r, sl] + wpe_v[r, sl]

        pltpu.async_copy(rows[b], out_hbm.at[pl.ds(base + g * _K, _K)],
                         sem_o[b])

    start_gather(0, 0)
    start_gather(1, 1)
    start_gather(2, 2)
    wait_gather(0)
    add_and_store(0, 0)
    start_gather(3, 3)
    wait_gather(1)
    add_and_store(1, 1)

    def steady(g, b):
        nb = (b + 2) % _NBUF
        wait_out(nb)
        start_gather(g + 2, nb)
        wait_gather(b)
        add_and_store(g, b)

    def quad_body(i, carry):
        g = _NBUF * i + 2
        for b in range(_NBUF):
            steady(g + b, (2 + b) % _NBUF)
        return carry

    lax.fori_loop(0, (_NCHUNK - 4) // _NBUF, quad_body, 0)

    wait_out(0)
    wait_gather(2)
    add_and_store(_NCHUNK - 2, 2)
    wait_out(1)
    wait_gather(3)
    add_and_store(_NCHUNK - 1, 3)
    wait_out(2)
    wait_out(3)


def kernel(input_ids, wte, wpe):
    ids_flat = input_ids.reshape(-1).astype(jnp.int32)
    out = _emb_lookup(ids_flat, wte, wpe)
    return out.reshape(_BATCH, _CHUNK, _N_EMBD)
